# baseline (speedup 1.0000x reference)
import functools
import jax
from jax.experimental.compute_on import compute_on
import jax.numpy as jnp
from jax import lax
from jax.experimental import pallas as pl
from jax.experimental.pallas import tpu as pltpu
from jax.experimental.pallas import tpu_sc as plsc

B, S, E = 4, 4096, 64
BETA_ENTROPY = 0.1
TEMP_LO, TEMP_HI = 0.5, 1.5
EPS = 1e-8
N_TOKENS = B * S

N_SC = 4096
N_TC = N_TOKENS - N_SC
TC_BLOCK = 4096
TC_GRID = N_TC // TC_BLOCK

NW = 32
TOK_PER_W = N_SC // NW
CHUNK = 128
N_CHUNKS = TOK_PER_W // CHUNK

LN2 = 0.6931471805599453
_LN_COEFFS = (1.0243828631e-02, -5.3267477734e-02, 1.3198966240e-01,
              -2.2396689943e-01, 3.2751171370e-01, -4.9933394898e-01,
              9.9997024330e-01, 2.2159764866e-07)


def _vln(x):
    xi = plsc.bitcast(x, jnp.int32)
    ef = ((xi >> 23) - 127).astype(jnp.float32)
    u = plsc.bitcast((xi & 0x7FFFFF) | 0x3F800000, jnp.float32) - 1.0
    p = jnp.full((16,), _LN_COEFFS[0], dtype=jnp.float32)
    for c in _LN_COEFFS[1:]:
        p = p * u + c
    return ef * LN2 + p


def _sc_body(tg_hbm, sg_hbm, t16_hbm, out_hbm, tg_v, sg_v, t_v, stage_v):
    wid = lax.axis_index("s") * 2 + lax.axis_index("c")
    base = wid * TOK_PER_W
    pltpu.sync_copy(t16_hbm, t_v)

    inv_t = 1.0 / t_v[...]
    lane = jax.lax.iota(jnp.int32, 16)
    zeros = jnp.zeros((16,), jnp.float32)

    acc_kl = zeros
    acc_ent = zeros
    for c in range(N_CHUNKS):
        pltpu.sync_copy(tg_hbm.at[pl.ds(base + c * CHUNK, CHUNK), :], tg_v)
        pltpu.sync_copy(sg_hbm.at[pl.ds(base + c * CHUNK, CHUNK), :], sg_v)

        def group_step(g, acc):
            a_kl, a_ent = acc
            tok_idx = g * 16 + lane

            def expert_step(i, cr):
                vsa, vsb, vnum, vent = cr
                for k in range(8):
                    e_idx = jnp.full((16,), k, jnp.int32) + i * 8
                    tv = plsc.load_gather(tg_v, [tok_idx, e_idx])
                    sv = plsc.load_gather(sg_v, [tok_idx, e_idx])
                    tl = _vln(tv + EPS)
                    sl = _vln(sv + EPS)
                    ea = jnp.exp(tl * inv_t)
                    eb = jnp.exp(sl * inv_t)
                    vsa = vsa + ea
                    vsb = vsb + eb
                    vnum = vnum + ea * (tl - sl)
                    vent = vent + sv * sl
                return (vsa, vsb, vnum, vent)

            vsa, vsb, vnum, vent = lax.fori_loop(
                0, E // 8, expert_step, (zeros, zeros, zeros, zeros))
            vkl = vnum * inv_t / vsa - _vln(vsa) + _vln(vsb)
            return (a_kl + vkl, a_ent + vent)

        acc_kl, acc_ent = lax.fori_loop(
            0, CHUNK // 16, group_step, (acc_kl, acc_ent))

    stage_v[pl.ds(0, 16)] = acc_kl
    stage_v[pl.ds(16, 16)] = acc_ent
    for k in range(2, 8):
        stage_v[pl.ds(16 * k, 16)] = zeros
    pltpu.sync_copy(stage_v, out_hbm.at[wid])


@functools.partial(
    pl.kernel,
    out_type=jax.ShapeDtypeStruct((NW, 128), jnp.float32),
    mesh=plsc.VectorSubcoreMesh(core_axis_name="c", subcore_axis_name="s"),
    scratch_types=[
        pltpu.VMEM((CHUNK, E), jnp.float32),
        pltpu.VMEM((CHUNK, E), jnp.float32),
        pltpu.VMEM((16,), jnp.float32),
        pltpu.VMEM((128,), jnp.float32),
    ],
    compiler_params=pltpu.CompilerParams(needs_layout_passes=False),
)
def _sc_loss(tg_hbm, sg_hbm, t16_hbm, out_hbm, tg_v, sg_v, t_v, stage_v):
    _sc_body(tg_hbm, sg_hbm, t16_hbm, out_hbm, tg_v, sg_v, t_v, stage_v)


def _tc_kernel(temp_ref, tg_ref, sg_ref, out_ref, acc_ref):
    i = pl.program_id(0)

    @pl.when(i == 0)
    def _init():
        acc_ref[0] = 0.0
        acc_ref[1] = 0.0

    T = jnp.clip(temp_ref[0], TEMP_LO, TEMP_HI)
    inv_T = 1.0 / T

    tg = tg_ref[...]
    sg = sg_ref[...]

    t_log = jnp.log(tg + EPS)
    s_log = jnp.log(sg + EPS)
    ea = jnp.exp(t_log * inv_T)
    eb = jnp.exp(s_log * inv_T)

    sa = jnp.sum(ea, axis=-1)
    sb = jnp.sum(eb, axis=-1)
    num = jnp.sum(ea * (t_log - s_log), axis=-1) * inv_T

    kl_rows = num / sa - jnp.log(sa) + jnp.log(sb)
    acc_ref[0] += jnp.sum(kl_rows)
    acc_ref[1] += jnp.sum(sg * s_log)

    @pl.when(i == TC_GRID - 1)
    def _finish():
        out_ref[0] = acc_ref[0]
        out_ref[1] = acc_ref[1]


def _tc_loss(tg, sg, temp):
    return pl.pallas_call(
        _tc_kernel,
        grid=(TC_GRID,),
        in_specs=[
            pl.BlockSpec(memory_space=pltpu.SMEM),
            pl.BlockSpec((TC_BLOCK, E), lambda i: (i + N_SC // TC_BLOCK, 0)),
            pl.BlockSpec((TC_BLOCK, E), lambda i: (i + N_SC // TC_BLOCK, 0)),
        ],
        out_specs=pl.BlockSpec(memory_space=pltpu.SMEM),
        out_shape=jax.ShapeDtypeStruct((2,), jnp.float32),
        scratch_shapes=[pltpu.SMEM((2,), jnp.float32)],
    )(temp, tg, sg)


def kernel(teacher_gates, student_gates, teacher_hidden_states, student_hidden_states, input_ids, temperature):
    tg = teacher_gates.reshape(N_TOKENS, E)
    sg = student_gates.reshape(N_TOKENS, E)
    T = jnp.clip(temperature, TEMP_LO, TEMP_HI)
    t16 = jnp.broadcast_to(T, (16,)).astype(jnp.float32)
    sc_call = compute_on("tpu_sparsecore")(jax.jit(_sc_loss))
    sc_part = sc_call(tg, sg, t16)
    tc_part = _tc_loss(tg, sg, temperature.reshape(1))
    kl_sum = jnp.sum(sc_part[:, :16]) + tc_part[0]
    ent_sum = jnp.sum(sc_part[:, 16:32]) + tc_part[1]
    inv_n = 1.0 / N_TOKENS
    return kl_sum * inv_n * (T * T) + BETA_ENTROPY * ent_sum * inv_n


# speedup vs baseline: 1.0043x; 1.0043x over previous
import functools
import jax
from jax.experimental.compute_on import compute_on
import jax.numpy as jnp
from jax import lax
from jax.experimental import pallas as pl
from jax.experimental.pallas import tpu as pltpu
from jax.experimental.pallas import tpu_sc as plsc

B, S, E = 4, 4096, 64
BETA_ENTROPY = 0.1
TEMP_LO, TEMP_HI = 0.5, 1.5
EPS = 1e-8
N_TOKENS = B * S

N_SC = 4096
N_TC = N_TOKENS - N_SC
TC_BLOCK = 4096
TC_GRID = N_TC // TC_BLOCK

NW = 32
TOK_PER_W = N_SC // NW
CHUNK = 128
N_CHUNKS = TOK_PER_W // CHUNK

LN2 = 0.6931471805599453
_LN_COEFFS = (1.0243828631e-02, -5.3267477734e-02, 1.3198966240e-01,
              -2.2396689943e-01, 3.2751171370e-01, -4.9933394898e-01,
              9.9997024330e-01, 2.2159764866e-07)


def _vln(x):
    xi = plsc.bitcast(x, jnp.int32)
    ef = ((xi >> 23) - 127).astype(jnp.float32)
    u = plsc.bitcast((xi & 0x7FFFFF) | 0x3F800000, jnp.float32) - 1.0
    p = jnp.full((16,), _LN_COEFFS[0], dtype=jnp.float32)
    for c in _LN_COEFFS[1:]:
        p = p * u + c
    return ef * LN2 + p


def _sc_body(tg_hbm, sg_hbm, t16_hbm, out_hbm, tg_v, sg_v, t_v, stage_v):
    wid = lax.axis_index("s") * 2 + lax.axis_index("c")
    base = wid * TOK_PER_W
    pltpu.sync_copy(t16_hbm, t_v)

    inv_t = 1.0 / t_v[...]
    lane = jax.lax.iota(jnp.int32, 16)
    zeros = jnp.zeros((16,), jnp.float32)

    acc_kl = zeros
    acc_ent = zeros
    for c in range(N_CHUNKS):
        pltpu.sync_copy(tg_hbm.at[pl.ds(base + c * CHUNK, CHUNK), :], tg_v)
        pltpu.sync_copy(sg_hbm.at[pl.ds(base + c * CHUNK, CHUNK), :], sg_v)

        def group_step(g, acc):
            a_kl, a_ent = acc
            tok_idx = g * 16 + lane

            def expert_step(i, cr):
                vsa, vsb, vnum, vent = cr
                for k in range(8):
                    e_idx = jnp.full((16,), k, jnp.int32) + i * 8
                    tv = plsc.load_gather(tg_v, [tok_idx, e_idx])
                    sv = plsc.load_gather(sg_v, [tok_idx, e_idx])
                    tl = _vln(tv + EPS)
                    sl = _vln(sv + EPS)
                    ea = jnp.exp(tl * inv_t)
                    eb = jnp.exp(sl * inv_t)
                    vsa = vsa + ea
                    vsb = vsb + eb
                    vnum = vnum + ea * (tl - sl)
                    vent = vent + sv * sl
                return (vsa, vsb, vnum, vent)

            vsa, vsb, vnum, vent = lax.fori_loop(
                0, E // 8, expert_step, (zeros, zeros, zeros, zeros))
            vkl = vnum * inv_t / vsa - _vln(vsa) + _vln(vsb)
            return (a_kl + vkl, a_ent + vent)

        acc_kl, acc_ent = lax.fori_loop(
            0, CHUNK // 16, group_step, (acc_kl, acc_ent))

    stage_v[pl.ds(0, 16)] = acc_kl
    stage_v[pl.ds(16, 16)] = acc_ent
    for k in range(2, 8):
        stage_v[pl.ds(16 * k, 16)] = zeros
    pltpu.sync_copy(stage_v, out_hbm.at[wid])


@functools.partial(
    pl.kernel,
    out_type=jax.ShapeDtypeStruct((NW, 128), jnp.float32),
    mesh=plsc.VectorSubcoreMesh(core_axis_name="c", subcore_axis_name="s"),
    scratch_types=[
        pltpu.VMEM((CHUNK, E), jnp.float32),
        pltpu.VMEM((CHUNK, E), jnp.float32),
        pltpu.VMEM((16,), jnp.float32),
        pltpu.VMEM((128,), jnp.float32),
    ],
    compiler_params=pltpu.CompilerParams(needs_layout_passes=False, skip_device_barrier=True),
)
def _sc_loss(tg_hbm, sg_hbm, t16_hbm, out_hbm, tg_v, sg_v, t_v, stage_v):
    _sc_body(tg_hbm, sg_hbm, t16_hbm, out_hbm, tg_v, sg_v, t_v, stage_v)


def _tc_kernel(temp_ref, tg_ref, sg_ref, out_ref, acc_ref):
    i = pl.program_id(0)

    @pl.when(i == 0)
    def _init():
        acc_ref[0] = 0.0
        acc_ref[1] = 0.0

    T = jnp.clip(temp_ref[0], TEMP_LO, TEMP_HI)
    inv_T = 1.0 / T

    tg = tg_ref[...]
    sg = sg_ref[...]

    t_log = jnp.log(tg + EPS)
    s_log = jnp.log(sg + EPS)
    ea = jnp.exp(t_log * inv_T)
    eb = jnp.exp(s_log * inv_T)

    sa = jnp.sum(ea, axis=-1)
    sb = jnp.sum(eb, axis=-1)
    num = jnp.sum(ea * (t_log - s_log), axis=-1) * inv_T

    kl_rows = num / sa - jnp.log(sa) + jnp.log(sb)
    acc_ref[0] += jnp.sum(kl_rows)
    acc_ref[1] += jnp.sum(sg * s_log)

    @pl.when(i == TC_GRID - 1)
    def _finish():
        out_ref[0] = acc_ref[0]
        out_ref[1] = acc_ref[1]


def _tc_loss(tg, sg, temp):
    return pl.pallas_call(
        _tc_kernel,
        grid=(TC_GRID,),
        in_specs=[
            pl.BlockSpec(memory_space=pltpu.SMEM),
            pl.BlockSpec((TC_BLOCK, E), lambda i: (i + N_SC // TC_BLOCK, 0)),
            pl.BlockSpec((TC_BLOCK, E), lambda i: (i + N_SC // TC_BLOCK, 0)),
        ],
        out_specs=pl.BlockSpec(memory_space=pltpu.SMEM),
        out_shape=jax.ShapeDtypeStruct((2,), jnp.float32),
        scratch_shapes=[pltpu.SMEM((2,), jnp.float32)],
    )(temp, tg, sg)


def kernel(teacher_gates, student_gates, teacher_hidden_states, student_hidden_states, input_ids, temperature):
    tg = teacher_gates.reshape(N_TOKENS, E)
    sg = student_gates.reshape(N_TOKENS, E)
    T = jnp.clip(temperature, TEMP_LO, TEMP_HI)
    t16 = jnp.broadcast_to(T, (16,)).astype(jnp.float32)
    sc_part = _sc_loss(tg, sg, t16)
    tc_part = _tc_loss(tg, sg, temperature.reshape(1))
    kl_sum = jnp.sum(sc_part[:, :16]) + tc_part[0]
    ent_sum = jnp.sum(sc_part[:, 16:32]) + tc_part[1]
    inv_n = 1.0 / N_TOKENS
    return kl_sum * inv_n * (T * T) + BETA_ENTROPY * ent_sum * inv_n


# speedup vs baseline: 1.0740x; 1.0694x over previous
import functools
import jax
from jax.experimental.compute_on import compute_on
import jax.numpy as jnp
from jax import lax
from jax.experimental import pallas as pl
from jax.experimental.pallas import tpu as pltpu
from jax.experimental.pallas import tpu_sc as plsc

B, S, E = 4, 4096, 64
BETA_ENTROPY = 0.1
TEMP_LO, TEMP_HI = 0.5, 1.5
EPS = 1e-8
N_TOKENS = B * S

N_SC = 512
N_TC = N_TOKENS - N_SC
TC_BLOCK = 7936
TC_GRID = N_TC // TC_BLOCK

NW = 32
TOK_PER_W = N_SC // NW
CHUNK = 16
N_CHUNKS = TOK_PER_W // CHUNK

LN2 = 0.6931471805599453
_LN_COEFFS = (1.0243828631e-02, -5.3267477734e-02, 1.3198966240e-01,
              -2.2396689943e-01, 3.2751171370e-01, -4.9933394898e-01,
              9.9997024330e-01, 2.2159764866e-07)


def _vln(x):
    xi = plsc.bitcast(x, jnp.int32)
    ef = ((xi >> 23) - 127).astype(jnp.float32)
    u = plsc.bitcast((xi & 0x7FFFFF) | 0x3F800000, jnp.float32) - 1.0
    p = jnp.full((16,), _LN_COEFFS[0], dtype=jnp.float32)
    for c in _LN_COEFFS[1:]:
        p = p * u + c
    return ef * LN2 + p


def _sc_body(tg_hbm, sg_hbm, t16_hbm, out_hbm, tg_v, sg_v, t_v, stage_v):
    wid = lax.axis_index("s") * 2 + lax.axis_index("c")
    base = N_TC + wid * TOK_PER_W
    pltpu.sync_copy(t16_hbm, t_v)

    inv_t = 1.0 / t_v[...]
    lane = jax.lax.iota(jnp.int32, 16)
    zeros = jnp.zeros((16,), jnp.float32)

    acc_kl = zeros
    acc_ent = zeros
    for c in range(N_CHUNKS):
        pltpu.sync_copy(tg_hbm.at[pl.ds(base + c * CHUNK, CHUNK), :], tg_v)
        pltpu.sync_copy(sg_hbm.at[pl.ds(base + c * CHUNK, CHUNK), :], sg_v)

        def group_step(g, acc):
            a_kl, a_ent = acc
            tok_idx = g * 16 + lane

            def expert_step(i, cr):
                vsa, vsb, vnum, vent = cr
                for k in range(8):
                    e_idx = jnp.full((16,), k, jnp.int32) + i * 8
                    tv = plsc.load_gather(tg_v, [tok_idx, e_idx])
                    sv = plsc.load_gather(sg_v, [tok_idx, e_idx])
                    tl = _vln(tv + EPS)
                    sl = _vln(sv + EPS)
                    ea = jnp.exp(tl * inv_t)
                    eb = jnp.exp(sl * inv_t)
                    vsa = vsa + ea
                    vsb = vsb + eb
                    vnum = vnum + ea * (tl - sl)
                    vent = vent + sv * sl
                return (vsa, vsb, vnum, vent)

            vsa, vsb, vnum, vent = lax.fori_loop(
                0, E // 8, expert_step, (zeros, zeros, zeros, zeros))
            vkl = vnum * inv_t / vsa - _vln(vsa) + _vln(vsb)
            return (a_kl + vkl, a_ent + vent)

        acc_kl, acc_ent = lax.fori_loop(
            0, CHUNK // 16, group_step, (acc_kl, acc_ent))

    stage_v[pl.ds(0, 16)] = acc_kl
    stage_v[pl.ds(16, 16)] = acc_ent
    for k in range(2, 8):
        stage_v[pl.ds(16 * k, 16)] = zeros
    pltpu.sync_copy(stage_v, out_hbm.at[wid])


@functools.partial(
    pl.kernel,
    out_type=jax.ShapeDtypeStruct((NW, 128), jnp.float32),
    mesh=plsc.VectorSubcoreMesh(core_axis_name="c", subcore_axis_name="s"),
    scratch_types=[
        pltpu.VMEM((CHUNK, E), jnp.float32),
        pltpu.VMEM((CHUNK, E), jnp.float32),
        pltpu.VMEM((16,), jnp.float32),
        pltpu.VMEM((128,), jnp.float32),
    ],
    compiler_params=pltpu.CompilerParams(needs_layout_passes=False, skip_device_barrier=True),
)
def _sc_loss(tg_hbm, sg_hbm, t16_hbm, out_hbm, tg_v, sg_v, t_v, stage_v):
    _sc_body(tg_hbm, sg_hbm, t16_hbm, out_hbm, tg_v, sg_v, t_v, stage_v)


def _tc_kernel(temp_ref, tg_ref, sg_ref, out_ref, acc_ref):
    i = pl.program_id(0)

    @pl.when(i == 0)
    def _init():
        acc_ref[0] = 0.0
        acc_ref[1] = 0.0

    T = jnp.clip(temp_ref[0], TEMP_LO, TEMP_HI)
    inv_T = 1.0 / T

    tg = tg_ref[...]
    sg = sg_ref[...]

    t_log = jnp.log(tg + EPS)
    s_log = jnp.log(sg + EPS)
    ea = jnp.exp(t_log * inv_T)
    eb = jnp.exp(s_log * inv_T)

    sa = jnp.sum(ea, axis=-1)
    sb = jnp.sum(eb, axis=-1)
    num = jnp.sum(ea * (t_log - s_log), axis=-1) * inv_T

    kl_rows = num / sa - jnp.log(sa) + jnp.log(sb)
    acc_ref[0] += jnp.sum(kl_rows)
    acc_ref[1] += jnp.sum(sg * s_log)

    @pl.when(i == TC_GRID - 1)
    def _finish():
        out_ref[0] = acc_ref[0]
        out_ref[1] = acc_ref[1]


def _tc_loss(tg, sg, temp):
    return pl.pallas_call(
        _tc_kernel,
        grid=(TC_GRID,),
        in_specs=[
            pl.BlockSpec(memory_space=pltpu.SMEM),
            pl.BlockSpec((TC_BLOCK, E), lambda i: (i, 0)),
            pl.BlockSpec((TC_BLOCK, E), lambda i: (i, 0)),
        ],
        out_specs=pl.BlockSpec(memory_space=pltpu.SMEM),
        out_shape=jax.ShapeDtypeStruct((2,), jnp.float32),
        scratch_shapes=[pltpu.SMEM((2,), jnp.float32)],
    )(temp, tg, sg)


def kernel(teacher_gates, student_gates, teacher_hidden_states, student_hidden_states, input_ids, temperature):
    tg = teacher_gates.reshape(N_TOKENS, E)
    sg = student_gates.reshape(N_TOKENS, E)
    T = jnp.clip(temperature, TEMP_LO, TEMP_HI)
    t16 = jnp.broadcast_to(T, (16,)).astype(jnp.float32)
    sc_part = _sc_loss(tg, sg, t16)
    tc_part = _tc_loss(tg, sg, temperature.reshape(1))
    kl_sum = jnp.sum(sc_part[:, :16]) + tc_part[0]
    ent_sum = jnp.sum(sc_part[:, 16:32]) + tc_part[1]
    inv_n = 1.0 / N_TOKENS
    return kl_sum * inv_n * (T * T) + BETA_ENTROPY * ent_sum * inv_n


# speedup vs baseline: 1.8659x; 1.7374x over previous
import jax
import jax.numpy as jnp
from jax.experimental import pallas as pl
from jax.experimental.pallas import tpu as pltpu

B, S, E = 4, 4096, 64
BETA_ENTROPY = 0.1
TEMP_LO, TEMP_HI = 0.5, 1.5
EPS = 1e-8

N_TOKENS = B * S
HALF = N_TOKENS // 2
BLOCK = 4096
GRID = HALF // BLOCK


def _partial_sums(tg, sg, inv_T):
    t_log = jnp.log(tg + EPS)
    s_log = jnp.log(sg + EPS)
    ea = jnp.exp(t_log * inv_T)
    eb = jnp.exp(s_log * inv_T)

    sa = jnp.sum(ea, axis=-1)
    sb = jnp.sum(eb, axis=-1)
    num = jnp.sum(ea * (t_log - s_log), axis=-1) * inv_T

    kl_rows = num / sa - jnp.log(sa) + jnp.log(sb)
    return jnp.sum(kl_rows), jnp.sum(sg * s_log)


def _loss_kernel(temp_ref, tg0_ref, tg1_ref, sg0_ref, sg1_ref, out_ref,
                 acc_ref):
    i = pl.program_id(0)

    @pl.when(i == 0)
    def _init():
        acc_ref[0] = 0.0
        acc_ref[1] = 0.0

    T = jnp.clip(temp_ref[0], TEMP_LO, TEMP_HI)
    inv_T = 1.0 / T

    kl0, ent0 = _partial_sums(tg0_ref[...], sg0_ref[...], inv_T)
    kl1, ent1 = _partial_sums(tg1_ref[...], sg1_ref[...], inv_T)
    acc_ref[0] += kl0 + kl1
    acc_ref[1] += ent0 + ent1

    @pl.when(i == GRID - 1)
    def _finish():
        inv_n = 1.0 / N_TOKENS
        kl_loss = acc_ref[0] * inv_n * (T * T)
        student_entropy = -acc_ref[1] * inv_n
        out_ref[0] = kl_loss - BETA_ENTROPY * student_entropy


def kernel(teacher_gates, student_gates, teacher_hidden_states, student_hidden_states, input_ids, temperature):
    tg = teacher_gates.reshape(N_TOKENS, E)
    sg = student_gates.reshape(N_TOKENS, E)
    temp = temperature.reshape(1)
    nblk = HALF // BLOCK
    bs0 = pl.BlockSpec((BLOCK, E), lambda i: (i, 0))
    bs1 = pl.BlockSpec((BLOCK, E), lambda i, _n=nblk: (i + _n, 0))

    out = pl.pallas_call(
        _loss_kernel,
        grid=(GRID,),
        in_specs=[pl.BlockSpec(memory_space=pltpu.SMEM), bs0, bs1, bs0, bs1],
        out_specs=pl.BlockSpec(memory_space=pltpu.SMEM),
        out_shape=jax.ShapeDtypeStruct((1,), jnp.float32),
        scratch_shapes=[pltpu.SMEM((2,), jnp.float32)],
    )(temp, tg, tg, sg, sg)
    return out[0]


# speedup vs baseline: 1.9227x; 1.0304x over previous
import jax
import jax.numpy as jnp
from jax.experimental import pallas as pl
from jax.experimental.pallas import tpu as pltpu

B, S, E = 4, 4096, 64
BETA_ENTROPY = 0.1
TEMP_LO, TEMP_HI = 0.5, 1.5
EPS = 1e-8

N_TOKENS = B * S
HALF = N_TOKENS // 2
BLOCK = 1024
GRID = HALF // BLOCK


def _partial_sums(tg, sg, inv_T):
    t_log = jnp.log(tg + EPS)
    s_log = jnp.log(sg + EPS)
    ea = jnp.exp(t_log * inv_T)
    eb = jnp.exp(s_log * inv_T)

    sa = jnp.sum(ea, axis=-1)
    sb = jnp.sum(eb, axis=-1)
    num = jnp.sum(ea * (t_log - s_log), axis=-1) * inv_T

    kl_rows = num / sa - jnp.log(sa) + jnp.log(sb)
    return jnp.sum(kl_rows), jnp.sum(sg * s_log)


def _loss_kernel(temp_ref, tg0_ref, tg1_ref, sg0_ref, sg1_ref, out_ref,
                 acc_ref):
    i = pl.program_id(0)

    @pl.when(i == 0)
    def _init():
        acc_ref[0] = 0.0
        acc_ref[1] = 0.0

    T = jnp.clip(temp_ref[0], TEMP_LO, TEMP_HI)
    inv_T = 1.0 / T

    kl0, ent0 = _partial_sums(tg0_ref[...], sg0_ref[...], inv_T)
    kl1, ent1 = _partial_sums(tg1_ref[...], sg1_ref[...], inv_T)
    acc_ref[0] += kl0 + kl1
    acc_ref[1] += ent0 + ent1

    @pl.when(i == GRID - 1)
    def _finish():
        inv_n = 1.0 / N_TOKENS
        kl_loss = acc_ref[0] * inv_n * (T * T)
        student_entropy = -acc_ref[1] * inv_n
        out_ref[0] = kl_loss - BETA_ENTROPY * student_entropy


def kernel(teacher_gates, student_gates, teacher_hidden_states, student_hidden_states, input_ids, temperature):
    tg = teacher_gates.reshape(N_TOKENS, E)
    sg = student_gates.reshape(N_TOKENS, E)
    temp = temperature.reshape(1)
    nblk = HALF // BLOCK
    bs0 = pl.BlockSpec((BLOCK, E), lambda i: (i, 0))
    bs1 = pl.BlockSpec((BLOCK, E), lambda i, _n=nblk: (i + _n, 0))

    out = pl.pallas_call(
        _loss_kernel,
        grid=(GRID,),
        in_specs=[pl.BlockSpec(memory_space=pltpu.SMEM), bs0, bs1, bs0, bs1],
        out_specs=pl.BlockSpec(memory_space=pltpu.SMEM),
        out_shape=jax.ShapeDtypeStruct((1,), jnp.float32),
        scratch_shapes=[pltpu.SMEM((2,), jnp.float32)],
    )(temp, tg, tg, sg, sg)
    return out[0]


# speedup vs baseline: 1.9395x; 1.0087x over previous
import jax
import jax.numpy as jnp
from jax.experimental import pallas as pl
from jax.experimental.pallas import tpu as pltpu

B, S, E = 4, 4096, 64
BETA_ENTROPY = 0.1
TEMP_LO, TEMP_HI = 0.5, 1.5
EPS = 1e-8

N_TOKENS = B * S
HALF = N_TOKENS // 2
BLOCK = 1024
GRID = HALF // BLOCK


def _partial_sums(tg, sg, inv_T):
    t_log = jnp.log(tg + EPS)
    s_log = jnp.log(sg + EPS)
    ea = jnp.exp(t_log * inv_T)
    eb = jnp.exp(s_log * inv_T)

    sa = jnp.sum(ea, axis=-1, keepdims=True)
    sb = jnp.sum(eb, axis=-1, keepdims=True)
    num = jnp.sum(ea * (t_log - s_log), axis=-1, keepdims=True) * inv_T

    kl_rows = num / sa - jnp.log(sa) + jnp.log(sb)
    return jnp.sum(kl_rows), jnp.sum(sg * s_log)


def _loss_kernel(temp_ref, tg0_ref, tg1_ref, sg0_ref, sg1_ref, out_ref,
                 acc_ref):
    i = pl.program_id(0)

    @pl.when(i == 0)
    def _init():
        acc_ref[0] = 0.0
        acc_ref[1] = 0.0

    T = jnp.clip(temp_ref[0], TEMP_LO, TEMP_HI)
    inv_T = 1.0 / T

    kl0, ent0 = _partial_sums(tg0_ref[...], sg0_ref[...], inv_T)
    kl1, ent1 = _partial_sums(tg1_ref[...], sg1_ref[...], inv_T)
    acc_ref[0] += kl0 + kl1
    acc_ref[1] += ent0 + ent1

    @pl.when(i == GRID - 1)
    def _finish():
        inv_n = 1.0 / N_TOKENS
        kl_loss = acc_ref[0] * inv_n * (T * T)
        student_entropy = -acc_ref[1] * inv_n
        out_ref[0] = kl_loss - BETA_ENTROPY * student_entropy


def kernel(teacher_gates, student_gates, teacher_hidden_states, student_hidden_states, input_ids, temperature):
    tg = teacher_gates.reshape(N_TOKENS, E)
    sg = student_gates.reshape(N_TOKENS, E)
    temp = temperature.reshape(1)
    nblk = HALF // BLOCK
    bs0 = pl.BlockSpec((BLOCK, E), lambda i: (i, 0))
    bs1 = pl.BlockSpec((BLOCK, E), lambda i, _n=nblk: (i + _n, 0))

    out = pl.pallas_call(
        _loss_kernel,
        grid=(GRID,),
        in_specs=[pl.BlockSpec(memory_space=pltpu.SMEM), bs0, bs1, bs0, bs1],
        out_specs=pl.BlockSpec(memory_space=pltpu.SMEM),
        out_shape=jax.ShapeDtypeStruct((1,), jnp.float32),
        scratch_shapes=[pltpu.SMEM((2,), jnp.float32)],
    )(temp, tg, tg, sg, sg)
    return out[0]


# speedup vs baseline: 1.9601x; 1.0106x over previous
import jax
import jax.numpy as jnp
from jax.experimental import pallas as pl
from jax.experimental.pallas import tpu as pltpu

B, S, E = 4, 4096, 64
BETA_ENTROPY = 0.1
TEMP_LO, TEMP_HI = 0.5, 1.5
EPS = 1e-8

N_TOKENS = B * S
QUARTER = N_TOKENS // 4
BLOCK = 1024
GRID = QUARTER // BLOCK


def _partial_sums(tg, sg, inv_T):
    t_log = jnp.log(tg + EPS)
    s_log = jnp.log(sg + EPS)
    ea = jnp.exp(t_log * inv_T)
    eb = jnp.exp(s_log * inv_T)

    sa = jnp.sum(ea, axis=-1, keepdims=True)
    sb = jnp.sum(eb, axis=-1, keepdims=True)
    num = jnp.sum(ea * (t_log - s_log), axis=-1, keepdims=True) * inv_T

    kl_rows = num / sa - jnp.log(sa) + jnp.log(sb)
    return jnp.sum(kl_rows), jnp.sum(sg * s_log)


def _loss_kernel(temp_ref, tg0_ref, tg1_ref, tg2_ref, tg3_ref,
                 sg0_ref, sg1_ref, sg2_ref, sg3_ref, out_ref, acc_ref):
    i = pl.program_id(0)

    @pl.when(i == 0)
    def _init():
        acc_ref[0] = 0.0
        acc_ref[1] = 0.0

    T = jnp.clip(temp_ref[0], TEMP_LO, TEMP_HI)
    inv_T = 1.0 / T

    kl0, ent0 = _partial_sums(tg0_ref[...], sg0_ref[...], inv_T)
    kl1, ent1 = _partial_sums(tg1_ref[...], sg1_ref[...], inv_T)
    kl2, ent2 = _partial_sums(tg2_ref[...], sg2_ref[...], inv_T)
    kl3, ent3 = _partial_sums(tg3_ref[...], sg3_ref[...], inv_T)
    acc_ref[0] += kl0 + kl1 + kl2 + kl3
    acc_ref[1] += ent0 + ent1 + ent2 + ent3

    @pl.when(i == GRID - 1)
    def _finish():
        inv_n = 1.0 / N_TOKENS
        kl_loss = acc_ref[0] * inv_n * (T * T)
        student_entropy = -acc_ref[1] * inv_n
        out_ref[0] = kl_loss - BETA_ENTROPY * student_entropy


def kernel(teacher_gates, student_gates, teacher_hidden_states, student_hidden_states, input_ids, temperature):
    tg = teacher_gates.reshape(N_TOKENS, E)
    sg = student_gates.reshape(N_TOKENS, E)
    temp = temperature.reshape(1)
    nblk = QUARTER // BLOCK
    bss = [pl.BlockSpec((BLOCK, E), lambda i, _n=nblk, _q=q: (i + _q * _n, 0))
           for q in range(4)]

    out = pl.pallas_call(
        _loss_kernel,
        grid=(GRID,),
        in_specs=[pl.BlockSpec(memory_space=pltpu.SMEM)] + bss + bss,
        out_specs=pl.BlockSpec(memory_space=pltpu.SMEM),
        out_shape=jax.ShapeDtypeStruct((1,), jnp.float32),
        scratch_shapes=[pltpu.SMEM((2,), jnp.float32)],
    )(temp, tg, tg, tg, tg, sg, sg, sg, sg)
    return out[0]
